# Initial kernel scaffold; baseline (speedup 1.0000x reference)
import jax
import jax.numpy as jnp
from jax.experimental import pallas as pl


def kernel(x, edge_index, edge_type, basis0, att_r0, att0, root0, bias0, ln_g0, ln_b0, basis1, att_r1, att1, root1, bias1, ln_g1, ln_b1):
    raise NotImplementedError("write your pallas kernel here")


# speedup vs baseline: 7.2964x; 7.2964x over previous
import functools

import jax
import jax.numpy as jnp
from jax import lax
from jax.experimental import pallas as pl
from jax.experimental.pallas import tpu as pltpu
from jax.experimental.pallas import tpu_sc as plsc

N = 10000
E = 320000
D = 128
R = 8
NC = 2
NS = 16
NW = NC * NS
EW = E // NW
CH1 = 2000
C2 = 80
NP = 10240
RPT = NP // NS
ND = 80

BLK = 400
NBLK = N // BLK
BLK2 = 2000


def _tc_pre_body(h_ref, w_ref, a_ref, hw_ref, st_ref):
    hb = h_ref[...]
    w = w_ref[0]
    out = jnp.dot(hb, w, preferred_element_type=jnp.float32)
    hw_ref[0] = out
    av = a_ref[0, 0]
    st_ref[0, 0, 0] = jnp.sum(out * av[None, :], axis=1)


_tc_pre = pl.pallas_call(
    _tc_pre_body,
    grid=(9, NBLK),
    in_specs=[
        pl.BlockSpec((BLK, D), lambda r, i: (i, 0)),
        pl.BlockSpec((1, D, D), lambda r, i: (r, 0, 0)),
        pl.BlockSpec((1, 1, D), lambda r, i: (r, 0, 0)),
    ],
    out_specs=[
        pl.BlockSpec((1, BLK, D), lambda r, i: (r, i, 0)),
        pl.BlockSpec((1, 1, 1, BLK), lambda r, i: (r, i, 0, 0)),
    ],
    out_shape=[
        jax.ShapeDtypeStruct((9, N, D), jnp.float32),
        jax.ShapeDtypeStruct((9, NBLK, 1, BLK), jnp.float32),
    ],
)


def _tc_post_body(sp_ref, rec_ref, hr_ref, b_ref, g_ref, bb_ref, o_ref):
    S = sp_ref[0] + sp_ref[1]
    aggr = S * rec_ref[...] + hr_ref[...] + b_ref[...]
    mu = jnp.mean(aggr, axis=1, keepdims=True)
    xc = aggr - mu
    var = jnp.mean(xc * xc, axis=1, keepdims=True)
    y = xc / jnp.sqrt(var + 1e-5) * g_ref[...] + bb_ref[...]
    o_ref[...] = jnp.tanh(y)


_tc_post = pl.pallas_call(
    _tc_post_body,
    grid=(N // BLK2,),
    in_specs=[
        pl.BlockSpec((2, BLK2, D), lambda i: (0, i, 0)),
        pl.BlockSpec((BLK2, D), lambda i: (i, 0)),
        pl.BlockSpec((BLK2, D), lambda i: (i, 0)),
        pl.BlockSpec((1, D), lambda i: (0, 0)),
        pl.BlockSpec((1, D), lambda i: (0, 0)),
        pl.BlockSpec((1, D), lambda i: (0, 0)),
    ],
    out_specs=pl.BlockSpec((BLK2, D), lambda i: (i, 0)),
    out_shape=jax.ShapeDtypeStruct((N, D), jnp.float32),
)


_mesh = plsc.VectorSubcoreMesh(
    core_axis_name="c", subcore_axis_name="s", num_cores=NC, num_subcores=NS)


@functools.partial(
    pl.kernel,
    out_type=[
        jax.ShapeDtypeStruct((E,), jnp.float32),
        jax.ShapeDtypeStruct((E,), jnp.int32),
    ],
    mesh=_mesh,
    compiler_params=pltpu.CompilerParams(needs_layout_passes=False),
    scratch_types=[
        pltpu.VMEM((9 * N,), jnp.float32),
        pltpu.VMEM((16,), jnp.float32),
        pltpu.VMEM((CH1,), jnp.int32),
        pltpu.VMEM((CH1,), jnp.int32),
        pltpu.VMEM((CH1,), jnp.int32),
        pltpu.VMEM((CH1,), jnp.float32),
        pltpu.VMEM((CH1,), jnp.int32),
    ],
)
def _sc_edge_alpha(src_hbm, dst_hbm, et_hbm, tab_hbm, amax_hbm,
                   ex_hbm, q_hbm,
                   tab_v, a_v, src_v, dst_v, et_v, ex_v, q_v):
    wid = lax.axis_index("s") * NC + lax.axis_index("c")
    base = wid * EW
    pltpu.sync_copy(tab_hbm, tab_v)
    pltpu.sync_copy(amax_hbm, a_v)
    av = a_v[...]

    def chunk(j, carry):
        off = base + j * CH1
        pltpu.sync_copy(src_hbm.at[pl.ds(off, CH1)], src_v)
        pltpu.sync_copy(dst_hbm.at[pl.ds(off, CH1)], dst_v)
        pltpu.sync_copy(et_hbm.at[pl.ds(off, CH1)], et_v)

        def step(i, c2):
            sl = pl.ds(i * 16, 16)
            srcv = src_v[sl]
            dstv = dst_v[sl]
            etv = et_v[sl]
            qv = etv * N + srcv
            ajv = plsc.load_gather(tab_v, [qv])
            aiv = plsc.load_gather(tab_v, [dstv + R * N])
            al = aiv + ajv
            al = jnp.where(al >= 0, al, 0.2 * al)
            cap = aiv + av
            cap = jnp.where(cap >= 0, cap, 0.2 * cap)
            ex_v[sl] = jnp.exp(al - cap)
            q_v[sl] = qv
            return c2

        lax.fori_loop(0, CH1 // 16, step, 0)
        pltpu.sync_copy(ex_v, ex_hbm.at[pl.ds(off, CH1)])
        pltpu.sync_copy(q_v, q_hbm.at[pl.ds(off, CH1)])
        return carry

    lax.fori_loop(0, EW // CH1, chunk, 0)


@functools.partial(
    pl.kernel,
    out_type=[
        jax.ShapeDtypeStruct((NC, NP, D), jnp.float32),
        jax.ShapeDtypeStruct((NC, ND, D), jnp.float32),
    ],
    mesh=_mesh,
    compiler_params=pltpu.CompilerParams(needs_layout_passes=False),
    scratch_types=[
        pltpu.VMEM_SHARED((NP, D), jnp.float32),
        pltpu.VMEM_SHARED((ND, D), jnp.float32),
        pltpu.VMEM((C2,), jnp.int32),
        pltpu.VMEM((C2,), jnp.int32),
        pltpu.VMEM((C2,), jnp.int32),
        pltpu.VMEM((C2,), jnp.int32),
        pltpu.VMEM((C2,), jnp.float32),
        pltpu.VMEM((C2, D), jnp.float32),
        pltpu.VMEM((C2, D), jnp.float32),
        pltpu.VMEM((C2, D), jnp.float32),
        pltpu.SemaphoreType.DMA,
    ],
)
def _sc_edge_aggr(hw_hbm, q_hbm, dst_hbm, ex_hbm, z_hbm, out_hbm, outd_hbm,
                  acc, acc_d, q_v, dst_v, dr_v, dp_v, ex_v,
                  rows_v, stage_v, stage_d, sem):
    cc = lax.axis_index("c")
    ss = lax.axis_index("s")
    base = (ss * NC + cc) * EW
    r0 = ss * RPT
    pltpu.sync_copy(z_hbm.at[pl.ds(r0, RPT)], acc.at[pl.ds(r0, RPT)])

    @pl.when(ss == 0)
    def _():
        pltpu.sync_copy(z_hbm.at[pl.ds(0, ND)], acc_d)

    plsc.subcore_barrier()
    lane = lax.iota(jnp.int32, 16)

    def chunk(j, carry):
        off = base + j * C2
        pltpu.sync_copy(q_hbm.at[pl.ds(off, C2)], q_v)
        pltpu.sync_copy(dst_hbm.at[pl.ds(off, C2)], dst_v)
        pltpu.sync_copy(ex_hbm.at[pl.ds(off, C2)], ex_v)
        pltpu.async_copy(hw_hbm.at[q_v], rows_v, sem).wait()

        def prep(g, c2):
            sl = pl.ds(g * 16, 16)
            dv = dst_v[sl]
            dr_v[sl] = lax.shift_right_logical(dv, 7)
            dp_v[sl] = jnp.bitwise_and(dv, 127)
            return c2

        lax.fori_loop(0, C2 // 16, prep, 0)

        def edge(e, c2):
            eb = jnp.full((16,), e, jnp.int32)
            exb = plsc.load_gather(ex_v, [eb])
            pb = plsc.load_gather(dp_v, [eb])
            for k in range(D // 16):
                ks = pl.ds(k * 16, 16)
                stage_v[e, ks] = rows_v[e, ks] * exb
                stage_d[e, ks] = jnp.where(lane + (k * 16) == pb, exb, 0.0)
            return c2

        lax.fori_loop(0, C2, edge, 0)
        pltpu.sync_copy(stage_v, acc.at[dst_v], add=True)
        pltpu.sync_copy(stage_d, acc_d.at[dr_v], add=True)
        return carry

    lax.fori_loop(0, EW // C2, chunk, 0)
    plsc.subcore_barrier()
    pltpu.sync_copy(acc.at[pl.ds(r0, RPT)], out_hbm.at[cc, pl.ds(r0, RPT)])

    @pl.when(ss == 0)
    def _():
        pltpu.sync_copy(acc_d, outd_hbm.at[cc])


def _layer(h, src, dst, et, basis, att_r, att, root, bias, ln_g, ln_b, z):
    Wall = jnp.concatenate(
        [jnp.einsum('rb,bij->rij', att_r, basis), root[None]], axis=0)
    avec = jnp.concatenate(
        [jnp.tile(att[:, D:], (R, 1)), att[:, :D]], axis=0)[:, None, :]
    hWall, stab = _tc_pre(h, Wall, avec)
    tab = stab.reshape(9 * N)
    amax16 = jnp.full((16,), jnp.max(stab[:R]), jnp.float32)
    ex, q = _sc_edge_alpha(src, dst, et, tab, amax16)
    spart, dpart = _sc_edge_aggr(hWall.reshape(9 * N, D), q, dst, ex, z)
    dvec = (dpart[0] + dpart[1]).reshape(ND * D)[:N]
    rec = jnp.broadcast_to((1.0 / (dvec + 1e-16))[:, None], (N, D))
    return _tc_post(spart, rec, hWall[R], bias[None], ln_g[None], ln_b[None])


def kernel(x, edge_index, edge_type,
           basis0, att_r0, att0, root0, bias0, ln_g0, ln_b0,
           basis1, att_r1, att1, root1, bias1, ln_g1, ln_b1):
    src = edge_index[0]
    dst = edge_index[1]
    z = jnp.zeros((NP, D), jnp.float32)
    h1 = _layer(x, src, dst, edge_type,
                basis0, att_r0, att0, root0, bias0, ln_g0, ln_b0, z)
    h2 = _layer(h1, src, dst, edge_type,
                basis1, att_r1, att1, root1, bias1, ln_g1, ln_b1, z)
    return (h2, h1, h2)


# speedup vs baseline: 8.2625x; 1.1324x over previous
import functools

import jax
import jax.numpy as jnp
from jax import lax
from jax.experimental import pallas as pl
from jax.experimental.pallas import tpu as pltpu
from jax.experimental.pallas import tpu_sc as plsc

N = 10000
E = 320000
D = 128
R = 8
NC = 2
NS = 16
NW = NC * NS
EW = E // NW
CH1 = 2000
C2 = 80
NP = 10240
RPT = NP // NS
ND = 80

BLK = 400
NBLK = N // BLK
BLK2 = 2000


def _tc_pre_body(h_ref, w_ref, a_ref, hw_ref, st_ref):
    hb = h_ref[...]
    w = w_ref[0]
    out = jnp.dot(hb, w, preferred_element_type=jnp.float32)
    hw_ref[0] = out
    av = a_ref[0, 0]
    st_ref[0, 0, 0] = jnp.sum(out * av[None, :], axis=1)


_tc_pre = pl.pallas_call(
    _tc_pre_body,
    grid=(9, NBLK),
    in_specs=[
        pl.BlockSpec((BLK, D), lambda r, i: (i, 0)),
        pl.BlockSpec((1, D, D), lambda r, i: (r, 0, 0)),
        pl.BlockSpec((1, 1, D), lambda r, i: (r, 0, 0)),
    ],
    out_specs=[
        pl.BlockSpec((1, BLK, D), lambda r, i: (r, i, 0)),
        pl.BlockSpec((1, 1, 1, BLK), lambda r, i: (r, i, 0, 0)),
    ],
    out_shape=[
        jax.ShapeDtypeStruct((9, N, D), jnp.float32),
        jax.ShapeDtypeStruct((9, NBLK, 1, BLK), jnp.float32),
    ],
)


def _tc_post_body(sp_ref, rec_ref, hr_ref, b_ref, g_ref, bb_ref, o_ref):
    S = sp_ref[0] + sp_ref[1]
    aggr = S * rec_ref[...] + hr_ref[...] + b_ref[...]
    mu = jnp.mean(aggr, axis=1, keepdims=True)
    xc = aggr - mu
    var = jnp.mean(xc * xc, axis=1, keepdims=True)
    y = xc / jnp.sqrt(var + 1e-5) * g_ref[...] + bb_ref[...]
    o_ref[...] = jnp.tanh(y)


_tc_post = pl.pallas_call(
    _tc_post_body,
    grid=(N // BLK2,),
    in_specs=[
        pl.BlockSpec((2, BLK2, D), lambda i: (0, i, 0)),
        pl.BlockSpec((BLK2, D), lambda i: (i, 0)),
        pl.BlockSpec((BLK2, D), lambda i: (i, 0)),
        pl.BlockSpec((1, D), lambda i: (0, 0)),
        pl.BlockSpec((1, D), lambda i: (0, 0)),
        pl.BlockSpec((1, D), lambda i: (0, 0)),
    ],
    out_specs=pl.BlockSpec((BLK2, D), lambda i: (i, 0)),
    out_shape=jax.ShapeDtypeStruct((N, D), jnp.float32),
)


_mesh = plsc.VectorSubcoreMesh(
    core_axis_name="c", subcore_axis_name="s", num_cores=NC, num_subcores=NS)


@functools.partial(
    pl.kernel,
    out_type=[
        jax.ShapeDtypeStruct((E,), jnp.float32),
        jax.ShapeDtypeStruct((E,), jnp.int32),
    ],
    mesh=_mesh,
    compiler_params=pltpu.CompilerParams(needs_layout_passes=False),
    scratch_types=[
        pltpu.VMEM((9 * N,), jnp.float32),
        pltpu.VMEM((16,), jnp.float32),
        pltpu.VMEM((CH1,), jnp.int32),
        pltpu.VMEM((CH1,), jnp.int32),
        pltpu.VMEM((CH1,), jnp.int32),
        pltpu.VMEM((CH1,), jnp.float32),
        pltpu.VMEM((CH1,), jnp.int32),
    ],
)
def _sc_edge_alpha(src_hbm, dst_hbm, et_hbm, tab_hbm, amax_hbm,
                   ex_hbm, q_hbm,
                   tab_v, a_v, src_v, dst_v, et_v, ex_v, q_v):
    wid = lax.axis_index("s") * NC + lax.axis_index("c")
    base = wid * EW
    pltpu.sync_copy(tab_hbm, tab_v)
    pltpu.sync_copy(amax_hbm, a_v)
    av = a_v[...]

    def chunk(j, carry):
        off = base + j * CH1
        pltpu.sync_copy(src_hbm.at[pl.ds(off, CH1)], src_v)
        pltpu.sync_copy(dst_hbm.at[pl.ds(off, CH1)], dst_v)
        pltpu.sync_copy(et_hbm.at[pl.ds(off, CH1)], et_v)

        def step(i, c2):
            sl = pl.ds(i * 16, 16)
            srcv = src_v[sl]
            dstv = dst_v[sl]
            etv = et_v[sl]
            qv = etv * N + srcv
            ajv = plsc.load_gather(tab_v, [qv])
            aiv = plsc.load_gather(tab_v, [dstv + R * N])
            al = aiv + ajv
            al = jnp.where(al >= 0, al, 0.2 * al)
            cap = aiv + av
            cap = jnp.where(cap >= 0, cap, 0.2 * cap)
            ex_v[sl] = jnp.exp(al - cap)
            q_v[sl] = qv
            return c2

        lax.fori_loop(0, CH1 // 16, step, 0)
        pltpu.sync_copy(ex_v, ex_hbm.at[pl.ds(off, CH1)])
        pltpu.sync_copy(q_v, q_hbm.at[pl.ds(off, CH1)])
        return carry

    lax.fori_loop(0, EW // CH1, chunk, 0)


@functools.partial(
    pl.kernel,
    out_type=[
        jax.ShapeDtypeStruct((NC, NP, D), jnp.float32),
        jax.ShapeDtypeStruct((NC, ND, D), jnp.float32),
    ],
    mesh=_mesh,
    compiler_params=pltpu.CompilerParams(needs_layout_passes=False),
    scratch_types=[
        pltpu.VMEM_SHARED((NP, D), jnp.float32),
        pltpu.VMEM_SHARED((ND, D), jnp.float32),
        pltpu.VMEM((C2,), jnp.int32),
        pltpu.VMEM((C2,), jnp.int32),
        pltpu.VMEM((C2, D), jnp.float32),
        pltpu.VMEM((C2, D), jnp.float32),
        pltpu.VMEM((C2,), jnp.int32),
        pltpu.VMEM((C2,), jnp.int32),
        pltpu.VMEM((C2,), jnp.int32),
        pltpu.VMEM((C2,), jnp.float32),
        pltpu.VMEM((C2, D), jnp.float32),
        pltpu.SemaphoreType.DMA,
        pltpu.SemaphoreType.DMA,
    ],
)
def _sc_edge_aggr(hw_hbm, q_hbm, dst_hbm, ex_hbm, z_hbm, out_hbm, outd_hbm,
                  acc, acc_d, q_v0, q_v1, rows_v0, rows_v1,
                  dst_v, dr_v, dp_v, ex_v, stage_d, gsem0, gsem1):
    cc = lax.axis_index("c")
    ss = lax.axis_index("s")
    base = (ss * NC + cc) * EW
    r0 = ss * RPT
    pltpu.sync_copy(z_hbm.at[pl.ds(r0, RPT)], acc.at[pl.ds(r0, RPT)])

    @pl.when(ss == 0)
    def _():
        pltpu.sync_copy(z_hbm.at[pl.ds(0, ND)], acc_d)

    plsc.subcore_barrier()
    lane = lax.iota(jnp.int32, 16)
    NCHK = EW // C2

    def chunk(j, q_c, rows_c, sem_c, q_n, rows_n, sem_n, issue_next):
        if issue_next:
            pltpu.sync_copy(q_hbm.at[pl.ds(base + (j + 1) * C2, C2)], q_n)
            pltpu.async_copy(hw_hbm.at[q_n], rows_n, sem_n)
        off = base + j * C2
        pltpu.sync_copy(dst_hbm.at[pl.ds(off, C2)], dst_v)
        pltpu.sync_copy(ex_hbm.at[pl.ds(off, C2)], ex_v)

        def prep(g, c2):
            sl = pl.ds(g * 16, 16)
            dv = dst_v[sl]
            dr_v[sl] = lax.shift_right_logical(dv, 7)
            dp_v[sl] = jnp.bitwise_and(dv, 127)
            return c2

        lax.fori_loop(0, C2 // 16, prep, 0)
        pltpu.make_async_copy(hw_hbm.at[q_c], rows_c, sem_c).wait()

        def edge(e, c2):
            eb = jnp.full((16,), e, jnp.int32)
            exb = plsc.load_gather(ex_v, [eb])
            pb = plsc.load_gather(dp_v, [eb])
            for k in range(D // 16):
                ks = pl.ds(k * 16, 16)
                rows_c[e, ks] = rows_c[e, ks] * exb
                stage_d[e, ks] = jnp.where(lane + (k * 16) == pb, exb, 0.0)
            return c2

        lax.fori_loop(0, C2, edge, 0)
        pltpu.sync_copy(rows_c, acc.at[dst_v], add=True)
        pltpu.sync_copy(stage_d, acc_d.at[dr_v], add=True)

    pltpu.sync_copy(q_hbm.at[pl.ds(base, C2)], q_v0)
    pltpu.async_copy(hw_hbm.at[q_v0], rows_v0, gsem0)

    def pair(k, carry):
        chunk(2 * k, q_v0, rows_v0, gsem0, q_v1, rows_v1, gsem1, True)
        chunk(2 * k + 1, q_v1, rows_v1, gsem1, q_v0, rows_v0, gsem0, True)
        return carry

    lax.fori_loop(0, (NCHK - 1) // 2, pair, 0)
    chunk(NCHK - 1, q_v0, rows_v0, gsem0, q_v1, rows_v1, gsem1, False)
    plsc.subcore_barrier()
    pltpu.sync_copy(acc.at[pl.ds(r0, RPT)], out_hbm.at[cc, pl.ds(r0, RPT)])

    @pl.when(ss == 0)
    def _():
        pltpu.sync_copy(acc_d, outd_hbm.at[cc])


def _layer(h, src, dst, et, basis, att_r, att, root, bias, ln_g, ln_b, z):
    Wall = jnp.concatenate(
        [jnp.einsum('rb,bij->rij', att_r, basis), root[None]], axis=0)
    avec = jnp.concatenate(
        [jnp.tile(att[:, D:], (R, 1)), att[:, :D]], axis=0)[:, None, :]
    hWall, stab = _tc_pre(h, Wall, avec)
    tab = stab.reshape(9 * N)
    amax16 = jnp.full((16,), jnp.max(stab[:R]), jnp.float32)
    ex, q = _sc_edge_alpha(src, dst, et, tab, amax16)
    spart, dpart = _sc_edge_aggr(hWall.reshape(9 * N, D), q, dst, ex, z)
    dvec = (dpart[0] + dpart[1]).reshape(ND * D)[:N]
    rec = jnp.broadcast_to((1.0 / (dvec + 1e-16))[:, None], (N, D))
    return _tc_post(spart, rec, hWall[R], bias[None], ln_g[None], ln_b[None])


def kernel(x, edge_index, edge_type,
           basis0, att_r0, att0, root0, bias0, ln_g0, ln_b0,
           basis1, att_r1, att1, root1, bias1, ln_g1, ln_b1):
    src = edge_index[0]
    dst = edge_index[1]
    z = jnp.zeros((NP, D), jnp.float32)
    h1 = _layer(x, src, dst, edge_type,
                basis0, att_r0, att0, root0, bias0, ln_g0, ln_b0, z)
    h2 = _layer(h1, src, dst, edge_type,
                basis1, att_r1, att1, root1, bias1, ln_g1, ln_b1, z)
    return (h2, h1, h2)


# speedup vs baseline: 11.0115x; 1.3327x over previous
import functools

import jax
import jax.numpy as jnp
from jax import lax
from jax.experimental import pallas as pl
from jax.experimental.pallas import tpu as pltpu
from jax.experimental.pallas import tpu_sc as plsc

N = 10000
E = 320000
D = 128
R = 8
NC = 2
NS = 16
NW = NC * NS
EW = E // NW
CH1 = 2000
C2 = 80
C2P = C2 + 16
NP = 10240
RPT = NP // NS
ND = 80

BLK = 400
NBLK = N // BLK
BLK2 = 2000


def _tc_pre_body(h_ref, w_ref, a_ref, hw_ref, st_ref):
    hb = h_ref[...]
    w = w_ref[0]
    out = jnp.dot(hb, w, preferred_element_type=jnp.float32)
    hw_ref[0] = out
    av = a_ref[0, 0]
    st_ref[0, 0, 0] = jnp.sum(out * av[None, :], axis=1)


_tc_pre = pl.pallas_call(
    _tc_pre_body,
    grid=(9, NBLK),
    in_specs=[
        pl.BlockSpec((BLK, D), lambda r, i: (i, 0)),
        pl.BlockSpec((1, D, D), lambda r, i: (r, 0, 0)),
        pl.BlockSpec((1, 1, D), lambda r, i: (r, 0, 0)),
    ],
    out_specs=[
        pl.BlockSpec((1, BLK, D), lambda r, i: (r, i, 0)),
        pl.BlockSpec((1, 1, 1, BLK), lambda r, i: (r, i, 0, 0)),
    ],
    out_shape=[
        jax.ShapeDtypeStruct((9, N, D), jnp.float32),
        jax.ShapeDtypeStruct((9, NBLK, 1, BLK), jnp.float32),
    ],
)


def _tc_post_body(sp_ref, rec_ref, hr_ref, b_ref, g_ref, bb_ref, o_ref):
    S = sp_ref[0] + sp_ref[1]
    aggr = S * rec_ref[...] + hr_ref[...] + b_ref[...]
    mu = jnp.mean(aggr, axis=1, keepdims=True)
    xc = aggr - mu
    var = jnp.mean(xc * xc, axis=1, keepdims=True)
    y = xc / jnp.sqrt(var + 1e-5) * g_ref[...] + bb_ref[...]
    o_ref[...] = jnp.tanh(y)


_tc_post = pl.pallas_call(
    _tc_post_body,
    grid=(N // BLK2,),
    in_specs=[
        pl.BlockSpec((2, BLK2, D), lambda i: (0, i, 0)),
        pl.BlockSpec((BLK2, D), lambda i: (i, 0)),
        pl.BlockSpec((BLK2, D), lambda i: (i, 0)),
        pl.BlockSpec((1, D), lambda i: (0, 0)),
        pl.BlockSpec((1, D), lambda i: (0, 0)),
        pl.BlockSpec((1, D), lambda i: (0, 0)),
    ],
    out_specs=pl.BlockSpec((BLK2, D), lambda i: (i, 0)),
    out_shape=jax.ShapeDtypeStruct((N, D), jnp.float32),
)


_mesh = plsc.VectorSubcoreMesh(
    core_axis_name="c", subcore_axis_name="s", num_cores=NC, num_subcores=NS)


@functools.partial(
    pl.kernel,
    out_type=[
        jax.ShapeDtypeStruct((E,), jnp.float32),
        jax.ShapeDtypeStruct((E,), jnp.int32),
    ],
    mesh=_mesh,
    compiler_params=pltpu.CompilerParams(needs_layout_passes=False),
    scratch_types=[
        pltpu.VMEM((9 * N,), jnp.float32),
        pltpu.VMEM((16,), jnp.float32),
        pltpu.VMEM((CH1,), jnp.int32),
        pltpu.VMEM((CH1,), jnp.int32),
        pltpu.VMEM((CH1,), jnp.int32),
        pltpu.VMEM((CH1,), jnp.float32),
        pltpu.VMEM((CH1,), jnp.int32),
    ],
)
def _sc_edge_alpha(src_hbm, dst_hbm, et_hbm, tab_hbm, amax_hbm,
                   ex_hbm, q_hbm,
                   tab_v, a_v, src_v, dst_v, et_v, ex_v, q_v):
    wid = lax.axis_index("s") * NC + lax.axis_index("c")
    base = wid * EW
    pltpu.sync_copy(tab_hbm, tab_v)
    pltpu.sync_copy(amax_hbm, a_v)
    av = a_v[...]

    def chunk(j, carry):
        off = base + j * CH1
        pltpu.sync_copy(src_hbm.at[pl.ds(off, CH1)], src_v)
        pltpu.sync_copy(dst_hbm.at[pl.ds(off, CH1)], dst_v)
        pltpu.sync_copy(et_hbm.at[pl.ds(off, CH1)], et_v)

        def step(i, c2):
            sl = pl.ds(i * 16, 16)
            srcv = src_v[sl]
            dstv = dst_v[sl]
            etv = et_v[sl]
            qv = etv * N + srcv
            ajv = plsc.load_gather(tab_v, [qv])
            aiv = plsc.load_gather(tab_v, [dstv + R * N])
            al = aiv + ajv
            al = jnp.where(al >= 0, al, 0.2 * al)
            cap = aiv + av
            cap = jnp.where(cap >= 0, cap, 0.2 * cap)
            ex_v[sl] = jnp.exp(al - cap)
            q_v[sl] = qv
            return c2

        lax.fori_loop(0, CH1 // 16, step, 0)
        pltpu.sync_copy(ex_v, ex_hbm.at[pl.ds(off, CH1)])
        pltpu.sync_copy(q_v, q_hbm.at[pl.ds(off, CH1)])
        return carry

    lax.fori_loop(0, EW // CH1, chunk, 0)


@functools.partial(
    pl.kernel,
    out_type=[
        jax.ShapeDtypeStruct((NC, NP, D), jnp.float32),
        jax.ShapeDtypeStruct((NC, ND, D), jnp.float32),
    ],
    mesh=_mesh,
    compiler_params=pltpu.CompilerParams(needs_layout_passes=False),
    scratch_types=[
        pltpu.VMEM_SHARED((NP, D), jnp.float32),
        pltpu.VMEM_SHARED((ND, D), jnp.float32),
        pltpu.VMEM((C2,), jnp.int32),
        pltpu.VMEM((C2,), jnp.int32),
        pltpu.VMEM((C2, D), jnp.float32),
        pltpu.VMEM((C2, D), jnp.float32),
        pltpu.VMEM((C2,), jnp.int32),
        pltpu.VMEM((C2,), jnp.int32),
        pltpu.VMEM((C2P,), jnp.int32),
        pltpu.VMEM((C2P,), jnp.float32),
        pltpu.VMEM((C2, D), jnp.float32),
        pltpu.VMEM((C2P,), jnp.int32),
        pltpu.SemaphoreType.DMA,
        pltpu.SemaphoreType.DMA,
    ],
)
def _sc_edge_aggr(hw_hbm, q_hbm, dst_hbm, ex_hbm, z_hbm, out_hbm, outd_hbm,
                  acc, acc_d, q_v0, q_v1, rows_v0, rows_v1,
                  dst_v, dr_v, dp_v, ex_v, stage_d, pc_v, gsem0, gsem1):
    cc = lax.axis_index("c")
    ss = lax.axis_index("s")
    base = (ss * NC + cc) * EW
    r0 = ss * RPT
    pltpu.sync_copy(z_hbm.at[pl.ds(r0, RPT)], acc.at[pl.ds(r0, RPT)])

    @pl.when(ss == 0)
    def _():
        pltpu.sync_copy(z_hbm.at[pl.ds(0, ND)], acc_d)

    lane = lax.iota(jnp.int32, 16)
    zero16 = jnp.zeros((16,), jnp.float32)

    def z0(e, c):
        for k in range(D // 16):
            stage_d[e, pl.ds(k * 16, 16)] = zero16
        return c

    lax.fori_loop(0, C2, z0, 0)

    def z1(g, c):
        pc_v[pl.ds(g * 16, 16)] = jnp.zeros((16,), jnp.int32)
        return c

    lax.fori_loop(0, C2P // 16, z1, 0)
    plsc.subcore_barrier()
    NCHK = EW // C2

    def chunk(j, q_c, rows_c, sem_c, q_n, rows_n, sem_n, issue_next):
        if issue_next:
            pltpu.sync_copy(q_hbm.at[pl.ds(base + (j + 1) * C2, C2)], q_n)
            pltpu.async_copy(hw_hbm.at[q_n], rows_n, sem_n)
        off = base + j * C2
        pltpu.sync_copy(dst_hbm.at[pl.ds(off, C2)], dst_v)
        pltpu.sync_copy(ex_hbm.at[pl.ds(off, C2)], ex_v.at[pl.ds(0, C2)])

        def prep(g, c2):
            sl = pl.ds(g * 16, 16)
            dv = dst_v[sl]
            dr_v[sl] = lax.shift_right_logical(dv, 7)
            dp_v[sl] = jnp.bitwise_and(dv, 127)
            return c2

        lax.fori_loop(0, C2 // 16, prep, 0)
        pltpu.make_async_copy(hw_hbm.at[q_c], rows_c, sem_c).wait()

        def edge(e2, c2):
            for u in range(2):
                e = e2 * 2 + u
                exs = ex_v[pl.ds(e, 16)][0]
                exb = jnp.full((16,), exs, jnp.float32)
                pv = dp_v[pl.ds(e, 16)][0]
                prev = pc_v[pl.ds(e, 16)][0]
                pcol = jnp.bitwise_and(pv, 112)
                pmb = jnp.full((16,), jnp.bitwise_and(pv, 15), jnp.int32)
                for k in range(D // 16):
                    ks = pl.ds(k * 16, 16)
                    rows_c[e, ks] = rows_c[e, ks] * exb
                stage_d[e, pl.ds(prev, 16)] = zero16
                stage_d[e, pl.ds(pcol, 16)] = jnp.where(lane == pmb, exb, 0.0)
                eb = jnp.full((16,), e, jnp.int32)
                plsc.store_scatter(pc_v, [eb], jnp.full((16,), pcol, jnp.int32))
            return c2

        lax.fori_loop(0, C2 // 2, edge, 0)
        pltpu.sync_copy(rows_c, acc.at[dst_v], add=True)
        pltpu.sync_copy(stage_d, acc_d.at[dr_v], add=True)

    pltpu.sync_copy(q_hbm.at[pl.ds(base, C2)], q_v0)
    pltpu.async_copy(hw_hbm.at[q_v0], rows_v0, gsem0)

    def pair(k, carry):
        chunk(2 * k, q_v0, rows_v0, gsem0, q_v1, rows_v1, gsem1, True)
        chunk(2 * k + 1, q_v1, rows_v1, gsem1, q_v0, rows_v0, gsem0, True)
        return carry

    lax.fori_loop(0, (NCHK - 1) // 2, pair, 0)
    chunk(NCHK - 1, q_v0, rows_v0, gsem0, q_v1, rows_v1, gsem1, False)
    plsc.subcore_barrier()
    pltpu.sync_copy(acc.at[pl.ds(r0, RPT)], out_hbm.at[cc, pl.ds(r0, RPT)])

    @pl.when(ss == 0)
    def _():
        pltpu.sync_copy(acc_d, outd_hbm.at[cc])


def _layer(h, src, dst, et, basis, att_r, att, root, bias, ln_g, ln_b, z):
    Wall = jnp.concatenate(
        [jnp.einsum('rb,bij->rij', att_r, basis), root[None]], axis=0)
    avec = jnp.concatenate(
        [jnp.tile(att[:, D:], (R, 1)), att[:, :D]], axis=0)[:, None, :]
    hWall, stab = _tc_pre(h, Wall, avec)
    tab = stab.reshape(9 * N)
    amax16 = jnp.full((16,), jnp.max(stab[:R]), jnp.float32)
    ex, q = _sc_edge_alpha(src, dst, et, tab, amax16)
    spart, dpart = _sc_edge_aggr(hWall.reshape(9 * N, D), q, dst, ex, z)
    dvec = (dpart[0] + dpart[1]).reshape(ND * D)[:N]
    rec = jnp.broadcast_to((1.0 / (dvec + 1e-16))[:, None], (N, D))
    return _tc_post(spart, rec, hWall[R], bias[None], ln_g[None], ln_b[None])


def kernel(x, edge_index, edge_type,
           basis0, att_r0, att0, root0, bias0, ln_g0, ln_b0,
           basis1, att_r1, att1, root1, bias1, ln_g1, ln_b1):
    src = edge_index[0]
    dst = edge_index[1]
    z = jnp.zeros((NP, D), jnp.float32)
    h1 = _layer(x, src, dst, edge_type,
                basis0, att_r0, att0, root0, bias0, ln_g0, ln_b0, z)
    h2 = _layer(h1, src, dst, edge_type,
                basis1, att_r1, att1, root1, bias1, ln_g1, ln_b1, z)
    return (h2, h1, h2)


# speedup vs baseline: 11.1549x; 1.0130x over previous
import functools

import jax
import jax.numpy as jnp
from jax import lax
from jax.experimental import pallas as pl
from jax.experimental.pallas import tpu as pltpu
from jax.experimental.pallas import tpu_sc as plsc

N = 10000
E = 320000
D = 128
R = 8
NC = 2
NS = 16
NW = NC * NS
EW = E // NW
CH1 = 2000
C2 = 80
C2P = C2 + 16
NP = 10240
RPT = NP // NS
ND = 80

BLK = 400
NBLK = N // BLK
BLK2 = 2000


def _tc_pre_body(h_ref, w_ref, a_ref, hw_ref, st_ref):
    hb = h_ref[...]
    w = w_ref[0]
    out = jnp.dot(hb, w, preferred_element_type=jnp.float32)
    hw_ref[0] = out
    av = a_ref[0, 0]
    st_ref[0, 0, 0] = jnp.sum(out * av[None, :], axis=1)


_tc_pre = pl.pallas_call(
    _tc_pre_body,
    grid=(9, NBLK),
    in_specs=[
        pl.BlockSpec((BLK, D), lambda r, i: (i, 0)),
        pl.BlockSpec((1, D, D), lambda r, i: (r, 0, 0)),
        pl.BlockSpec((1, 1, D), lambda r, i: (r, 0, 0)),
    ],
    out_specs=[
        pl.BlockSpec((1, BLK, D), lambda r, i: (r, i, 0)),
        pl.BlockSpec((1, 1, 1, BLK), lambda r, i: (r, i, 0, 0)),
    ],
    out_shape=[
        jax.ShapeDtypeStruct((9, N, D), jnp.float32),
        jax.ShapeDtypeStruct((9, NBLK, 1, BLK), jnp.float32),
    ],
)


def _tc_post_body(sp_ref, rec_ref, hr_ref, b_ref, g_ref, bb_ref, o_ref):
    S = sp_ref[0] + sp_ref[1]
    aggr = S * rec_ref[...] + hr_ref[...] + b_ref[...]
    mu = jnp.mean(aggr, axis=1, keepdims=True)
    xc = aggr - mu
    var = jnp.mean(xc * xc, axis=1, keepdims=True)
    y = xc / jnp.sqrt(var + 1e-5) * g_ref[...] + bb_ref[...]
    o_ref[...] = jnp.tanh(y)


_tc_post = pl.pallas_call(
    _tc_post_body,
    grid=(N // BLK2,),
    in_specs=[
        pl.BlockSpec((2, BLK2, D), lambda i: (0, i, 0)),
        pl.BlockSpec((BLK2, D), lambda i: (i, 0)),
        pl.BlockSpec((BLK2, D), lambda i: (i, 0)),
        pl.BlockSpec((1, D), lambda i: (0, 0)),
        pl.BlockSpec((1, D), lambda i: (0, 0)),
        pl.BlockSpec((1, D), lambda i: (0, 0)),
    ],
    out_specs=pl.BlockSpec((BLK2, D), lambda i: (i, 0)),
    out_shape=jax.ShapeDtypeStruct((N, D), jnp.float32),
)


_mesh = plsc.VectorSubcoreMesh(
    core_axis_name="c", subcore_axis_name="s", num_cores=NC, num_subcores=NS)


@functools.partial(
    pl.kernel,
    out_type=[
        jax.ShapeDtypeStruct((E,), jnp.float32),
        jax.ShapeDtypeStruct((E,), jnp.int32),
    ],
    mesh=_mesh,
    compiler_params=pltpu.CompilerParams(needs_layout_passes=False),
    scratch_types=[
        pltpu.VMEM((9 * N,), jnp.float32),
        pltpu.VMEM((16,), jnp.float32),
        pltpu.VMEM((CH1,), jnp.int32),
        pltpu.VMEM((CH1,), jnp.int32),
        pltpu.VMEM((CH1,), jnp.int32),
        pltpu.VMEM((CH1,), jnp.float32),
        pltpu.VMEM((CH1,), jnp.int32),
    ],
)
def _sc_edge_alpha(src_hbm, dst_hbm, et_hbm, tab_hbm, amax_hbm,
                   ex_hbm, q_hbm,
                   tab_v, a_v, src_v, dst_v, et_v, ex_v, q_v):
    wid = lax.axis_index("s") * NC + lax.axis_index("c")
    base = wid * EW
    pltpu.sync_copy(tab_hbm, tab_v)
    pltpu.sync_copy(amax_hbm, a_v)
    av = a_v[...]

    def chunk(j, carry):
        off = base + j * CH1
        pltpu.sync_copy(src_hbm.at[pl.ds(off, CH1)], src_v)
        pltpu.sync_copy(dst_hbm.at[pl.ds(off, CH1)], dst_v)
        pltpu.sync_copy(et_hbm.at[pl.ds(off, CH1)], et_v)

        def step(i, c2):
            sl = pl.ds(i * 16, 16)
            srcv = src_v[sl]
            dstv = dst_v[sl]
            etv = et_v[sl]
            qv = etv * N + srcv
            ajv = plsc.load_gather(tab_v, [qv])
            aiv = plsc.load_gather(tab_v, [dstv + R * N])
            al = aiv + ajv
            al = jnp.where(al >= 0, al, 0.2 * al)
            cap = aiv + av
            cap = jnp.where(cap >= 0, cap, 0.2 * cap)
            ex_v[sl] = jnp.exp(al - cap)
            q_v[sl] = qv
            return c2

        lax.fori_loop(0, CH1 // 16, step, 0)
        pltpu.sync_copy(ex_v, ex_hbm.at[pl.ds(off, CH1)])
        pltpu.sync_copy(q_v, q_hbm.at[pl.ds(off, CH1)])
        return carry

    lax.fori_loop(0, EW // CH1, chunk, 0)


@functools.partial(
    pl.kernel,
    out_type=[
        jax.ShapeDtypeStruct((NC, NP, D), jnp.float32),
        jax.ShapeDtypeStruct((NC, NP), jnp.float32),
    ],
    mesh=_mesh,
    compiler_params=pltpu.CompilerParams(needs_layout_passes=False),
    scratch_types=[
        pltpu.VMEM_SHARED((NP, D), jnp.float32),
        pltpu.VMEM_SHARED((NP,), jnp.float32),
        pltpu.VMEM((C2,), jnp.int32),
        pltpu.VMEM((C2,), jnp.int32),
        pltpu.VMEM((C2, D), jnp.float32),
        pltpu.VMEM((C2, D), jnp.float32),
        pltpu.VMEM((C2,), jnp.int32),
        pltpu.VMEM((C2P,), jnp.int32),
        pltpu.VMEM((C2P,), jnp.float32),
        pltpu.VMEM((NP,), jnp.float32),
        pltpu.VMEM((NP,), jnp.int32),
        pltpu.SemaphoreType.DMA,
        pltpu.SemaphoreType.DMA,
    ],
)
def _sc_edge_aggr(hw_hbm, q_hbm, dst_hbm, ex_hbm, z_hbm, out_hbm, outd_hbm,
                  acc, acc_d, q_v0, q_v1, rows_v0, rows_v1,
                  dst_v, dstp_v, ex_v, dtab, idx_v, gsem0, gsem1):
    cc = lax.axis_index("c")
    ss = lax.axis_index("s")
    base = (ss * NC + cc) * EW
    r0 = ss * RPT
    pltpu.sync_copy(z_hbm.at[pl.ds(r0, RPT)], acc.at[pl.ds(r0, RPT)])

    lane = lax.iota(jnp.int32, 16)
    zero16 = jnp.zeros((16,), jnp.float32)
    mask0 = lane == 0

    def z0(g, c):
        dtab[pl.ds(g * 16, 16)] = zero16
        idx_v[pl.ds(g * 16, 16)] = lane + g * 16
        return c

    lax.fori_loop(0, NP // 16, z0, 0)
    pltpu.sync_copy(dtab.at[pl.ds(r0, RPT)], acc_d.at[pl.ds(r0, RPT)])
    plsc.subcore_barrier()
    NCHK = EW // C2

    def chunk(j, q_c, rows_c, sem_c, q_n, rows_n, sem_n, issue_next):
        if issue_next:
            pltpu.sync_copy(q_hbm.at[pl.ds(base + (j + 1) * C2, C2)], q_n)
            pltpu.async_copy(hw_hbm.at[q_n], rows_n, sem_n)
        off = base + j * C2
        pltpu.sync_copy(dst_hbm.at[pl.ds(off, C2)], dst_v)
        pltpu.sync_copy(dst_hbm.at[pl.ds(off, C2)], dstp_v.at[pl.ds(0, C2)])
        pltpu.sync_copy(ex_hbm.at[pl.ds(off, C2)], ex_v.at[pl.ds(0, C2)])
        pltpu.make_async_copy(hw_hbm.at[q_c], rows_c, sem_c).wait()

        def edge(e2, c2):
            for u in range(2):
                e = e2 * 2 + u
                exs = ex_v[pl.ds(e, 16)][0]
                exb = jnp.full((16,), exs, jnp.float32)
                d = dstp_v[pl.ds(e, 16)][0]
                for k in range(D // 16):
                    ks = pl.ds(k * 16, 16)
                    rows_c[e, ks] = rows_c[e, ks] * exb
                dsl = pl.ds(d, 16)
                dtab[dsl] = dtab[dsl] + jnp.where(mask0, exb, zero16)
            return c2

        lax.fori_loop(0, C2 // 2, edge, 0)
        pltpu.sync_copy(rows_c, acc.at[dst_v], add=True)

    pltpu.sync_copy(q_hbm.at[pl.ds(base, C2)], q_v0)
    pltpu.async_copy(hw_hbm.at[q_v0], rows_v0, gsem0)

    def pair(k, carry):
        chunk(2 * k, q_v0, rows_v0, gsem0, q_v1, rows_v1, gsem1, True)
        chunk(2 * k + 1, q_v1, rows_v1, gsem1, q_v0, rows_v0, gsem0, True)
        return carry

    lax.fori_loop(0, (NCHK - 1) // 2, pair, 0)
    chunk(NCHK - 1, q_v0, rows_v0, gsem0, q_v1, rows_v1, gsem1, False)
    pltpu.sync_copy(dtab, acc_d.at[idx_v], add=True)
    plsc.subcore_barrier()
    pltpu.sync_copy(acc.at[pl.ds(r0, RPT)], out_hbm.at[cc, pl.ds(r0, RPT)])
    pltpu.sync_copy(acc_d.at[pl.ds(r0, RPT)], outd_hbm.at[cc, pl.ds(r0, RPT)])


def _layer(h, src, dst, et, basis, att_r, att, root, bias, ln_g, ln_b, z):
    Wall = jnp.concatenate(
        [jnp.einsum('rb,bij->rij', att_r, basis), root[None]], axis=0)
    avec = jnp.concatenate(
        [jnp.tile(att[:, D:], (R, 1)), att[:, :D]], axis=0)[:, None, :]
    hWall, stab = _tc_pre(h, Wall, avec)
    tab = stab.reshape(9 * N)
    amax16 = jnp.full((16,), jnp.max(stab[:R]), jnp.float32)
    ex, q = _sc_edge_alpha(src, dst, et, tab, amax16)
    spart, dpart = _sc_edge_aggr(hWall.reshape(9 * N, D), q, dst, ex, z)
    dvec = (dpart[0] + dpart[1])[:N]
    rec = jnp.broadcast_to((1.0 / (dvec + 1e-16))[:, None], (N, D))
    return _tc_post(spart, rec, hWall[R], bias[None], ln_g[None], ln_b[None])


def kernel(x, edge_index, edge_type,
           basis0, att_r0, att0, root0, bias0, ln_g0, ln_b0,
           basis1, att_r1, att1, root1, bias1, ln_g1, ln_b1):
    src = edge_index[0]
    dst = edge_index[1]
    z = jnp.zeros((NP, D), jnp.float32)
    h1 = _layer(x, src, dst, edge_type,
                basis0, att_r0, att0, root0, bias0, ln_g0, ln_b0, z)
    h2 = _layer(h1, src, dst, edge_type,
                basis1, att_r1, att1, root1, bias1, ln_g1, ln_b1, z)
    return (h2, h1, h2)
